# baseline (speedup 1.0000x reference)
import functools

import jax
import jax.numpy as jnp
from jax import lax
from jax.experimental import pallas as pl
from jax.experimental.pallas import tpu as pltpu
from jax.experimental.pallas import tpu_sc as plsc

_BATCH = 16384
_HIST = 50
_DIM = 64
_N = _BATCH * _HIST
_NC = 2
_NS = 16
_NW = _NC * _NS
_PER_W = _N // _NW
_K = 128
_NCHUNK = _PER_W // _K
_NBUF = 4
_LEAD = 2
_NGRP = _NCHUNK // _NBUF


def _make_gather():
    mesh = plsc.VectorSubcoreMesh(core_axis_name="c", subcore_axis_name="s")

    @functools.partial(
        pl.kernel,
        mesh=mesh,
        out_type=jax.ShapeDtypeStruct((_N, _DIM), jnp.bfloat16),
        compiler_params=pltpu.CompilerParams(use_tc_tiling_on_sc=False),
        scratch_types=[
            pltpu.VMEM((_NCHUNK, _K), jnp.int32),
            pltpu.VMEM((_NBUF, _K, _DIM), jnp.bfloat16),
        ]
        + [pltpu.SemaphoreType.DMA] * (2 * _NBUF),
    )
    def gather_kernel(idx_hbm, table_hbm, out_hbm, idx_v, rows_v, *sems):
        gsem = sems[:_NBUF]
        osem = sems[_NBUF:]
        wid = lax.axis_index("s") * _NC + lax.axis_index("c")
        base = wid * _PER_W
        pltpu.sync_copy(idx_hbm.at[wid], idx_v)

        def start_gather(j, b):
            pltpu.async_copy(table_hbm.at[idx_v.at[j]], rows_v.at[b], gsem[b])

        def wait_gather(b):
            pltpu.make_async_copy(
                table_hbm.at[pl.ds(0, _K)], rows_v.at[b], gsem[b]
            ).wait()

        def wait_out(b):
            pltpu.make_async_copy(
                rows_v.at[b], out_hbm.at[pl.ds(base, _K)], osem[b]
            ).wait()

        for jj in range(_LEAD):
            start_gather(jj, jj)

        def body(g, carry):
            for b in range(_NBUF):
                j = g * _NBUF + b
                tgt = (b + _LEAD) % _NBUF
                jg = j + _LEAD

                @pl.when(jg < _NCHUNK)
                def _issue():
                    @pl.when(jg >= _NBUF)
                    def _reclaim():
                        wait_out(tgt)

                    start_gather(jg, tgt)

                wait_gather(b)
                pltpu.async_copy(
                    rows_v.at[b], out_hbm.at[pl.ds(base + j * _K, _K)], osem[b]
                )
            return carry

        lax.fori_loop(0, _NGRP, body, 0)
        for b in range(_NBUF):
            wait_out(b)

    return gather_kernel


_gather = _make_gather()


def kernel(wordids, table):
    idx = wordids.reshape(_NW, _NCHUNK, _K)
    if idx.dtype != jnp.int32:
        idx = idx.astype(jnp.int32)
    out = _gather(idx, table.astype(jnp.bfloat16))
    return out.astype(jnp.float32).reshape(_BATCH, _HIST, _DIM)


# speedup vs baseline: 1.0653x; 1.0653x over previous
import functools

import jax
import jax.numpy as jnp
from jax import lax
from jax.experimental import pallas as pl
from jax.experimental.pallas import tpu as pltpu
from jax.experimental.pallas import tpu_sc as plsc

_BATCH = 16384
_HIST = 50
_DIM = 64
_NC = 2
_NS = 16
_NW = _NC * _NS
_BBLK = _BATCH // _NW
_K = 128
_CPH = _BBLK // _K
_NCHUNK = _HIST * _CPH
_GBUF = 3
_LEAD = 2


def _make_gather():
    mesh = plsc.VectorSubcoreMesh(core_axis_name="c", subcore_axis_name="s")

    @functools.partial(
        pl.kernel,
        mesh=mesh,
        out_type=jax.ShapeDtypeStruct((_HIST, _DIM, _BATCH), jnp.float32),
        compiler_params=pltpu.CompilerParams(use_tc_tiling_on_sc=False, needs_layout_passes=False
        ),
        scratch_types=[
            pltpu.VMEM((_HIST * _BBLK,), jnp.int32),
            pltpu.VMEM((_GBUF, _K, _DIM), jnp.float32),
            pltpu.VMEM((2, _DIM, _BBLK), jnp.float32),
        ]
        + [pltpu.SemaphoreType.DMA] * (_GBUF + 2),
    )
    def gather_kernel(idx_hbm, table_hbm, out_hbm, idx_v, gbuf, oacc, *sems):
        gsem = sems[:_GBUF]
        osem = sems[_GBUF:]
        wid = lax.axis_index("s") * _NC + lax.axis_index("c")
        b0 = wid * _BBLK
        for hh in range(_HIST):
            pltpu.sync_copy(
                idx_hbm.at[hh, pl.ds(b0, _BBLK)], idx_v.at[pl.ds(hh * _BBLK, _BBLK)]
            )

        iota = lax.iota(jnp.int32, 16)

        def start_gather(t, g):
            pltpu.async_copy(
                table_hbm.at[idx_v.at[pl.ds(t * _K, _K)]], gbuf.at[g], gsem[g]
            )

        def wait_gather(g):
            pltpu.make_async_copy(
                table_hbm.at[pl.ds(0, _K)], gbuf.at[g], gsem[g]
            ).wait()

        def wait_out(p):
            pltpu.make_async_copy(
                oacc.at[p], out_hbm.at[0, :, pl.ds(b0, _BBLK)], osem[p]
            ).wait()

        def transpose_chunk(g, p, c):
            cbase = c * _K

            def row_body(b, carry):
                col = jnp.full((16,), cbase + b, jnp.int32)
                for d16 in range(_DIM // 16):
                    v = gbuf[g, b, pl.ds(d16 * 16, 16)]
                    plsc.store_scatter(oacc.at[p], [d16 * 16 + iota, col], v)
                return carry

            lax.fori_loop(0, _K, row_body, 0)

        for t in range(_LEAD):
            start_gather(t, t)

        def body(t, carry):
            g = lax.rem(t, _GBUF)
            h = t // _CPH
            c = t - h * _CPH
            p = lax.rem(h, 2)

            gi = lax.rem(t + _LEAD, _GBUF)

            @pl.when(t + _LEAD < _NCHUNK)
            def _issue():
                for gg in range(_GBUF):
                    @pl.when(gi == gg)
                    def _(gg=gg):
                        start_gather(t + _LEAD, gg)

            @pl.when((c == 0) & (h >= 2))
            def _reclaim():
                _wait_out_dyn(p)

            _wait_gather_dyn(g)
            _transpose_dyn(g, p, c)

            @pl.when(c == _CPH - 1)
            def _flush():
                _start_out_dyn(h, p)

            return carry

        def _wait_gather_dyn(g):
            for gg in range(_GBUF):
                @pl.when(g == gg)
                def _(gg=gg):
                    wait_gather(gg)

        def _transpose_dyn(g, p, c):
            for gg in range(_GBUF):
                for pp in range(2):
                    @pl.when((g == gg) & (p == pp))
                    def _(gg=gg, pp=pp):
                        transpose_chunk(gg, pp, c)

        def _start_out_dyn(h, p):
            for pp in range(2):
                @pl.when(p == pp)
                def _(pp=pp):
                    pltpu.async_copy(
                        oacc.at[pp], out_hbm.at[h, :, pl.ds(b0, _BBLK)], osem[pp]
                    )

        def _wait_out_dyn(p):
            for pp in range(2):
                @pl.when(p == pp)
                def _(pp=pp):
                    wait_out(pp)

        lax.fori_loop(0, _NCHUNK, body, 0)
        for pp in range(2):
            wait_out(pp)

    return gather_kernel


_gather = _make_gather()


def kernel(wordids, table):
    idx = wordids.T
    if idx.dtype != jnp.int32:
        idx = idx.astype(jnp.int32)
    out = _gather(idx, table)
    return out.transpose(2, 0, 1)


# speedup vs baseline: 1.5643x; 1.4685x over previous
import functools

import jax
import jax.numpy as jnp
from jax import lax
from jax.experimental import pallas as pl
from jax.experimental.pallas import tpu as pltpu
from jax.experimental.pallas import tpu_sc as plsc

_BATCH = 16384
_HIST = 50
_DIM = 64
_N = _BATCH * _HIST
_NC = 2
_NS = 16
_NW = _NC * _NS
_PER_W = _N // _NW
_K = 128
_NCHUNK = _PER_W // _K
_NBUF = 8
_LEAD = 4
_NGRP = _NCHUNK // _NBUF


def _make_gather():
    mesh = plsc.VectorSubcoreMesh(core_axis_name="c", subcore_axis_name="s")

    @functools.partial(
        pl.kernel,
        mesh=mesh,
        out_type=jax.ShapeDtypeStruct((_N, _DIM), jnp.float32),
        compiler_params=pltpu.CompilerParams(use_tc_tiling_on_sc=False),
        scratch_types=[
            pltpu.VMEM((_NCHUNK, _K), jnp.int32),
            pltpu.VMEM((_NBUF, _K, _DIM), jnp.float32),
        ]
        + [pltpu.SemaphoreType.DMA] * (2 * _NBUF),
    )
    def gather_kernel(idx_hbm, table_hbm, out_hbm, idx_v, rows_v, *sems):
        gsem = sems[:_NBUF]
        osem = sems[_NBUF:]
        wid = lax.axis_index("s") * _NC + lax.axis_index("c")
        base = wid * _PER_W
        pltpu.sync_copy(idx_hbm.at[wid], idx_v)

        def start_gather(j, b):
            pltpu.async_copy(table_hbm.at[idx_v.at[j]], rows_v.at[b], gsem[b])

        def wait_gather(b):
            pltpu.make_async_copy(
                out_hbm.at[pl.ds(base, _K)], rows_v.at[b], gsem[b]
            ).wait()

        def wait_out(b):
            pltpu.make_async_copy(
                rows_v.at[b], out_hbm.at[pl.ds(base, _K)], osem[b]
            ).wait()

        for jj in range(_LEAD):
            start_gather(jj, jj)

        def body(g, carry):
            for b in range(_NBUF):
                j = g * _NBUF + b
                tgt = (b + _LEAD) % _NBUF
                jg = j + _LEAD

                @pl.when(jg < _NCHUNK)
                def _issue():
                    @pl.when(jg >= _NBUF)
                    def _reclaim():
                        wait_out(tgt)

                    start_gather(jg, tgt)

                wait_gather(b)
                pltpu.async_copy(
                    rows_v.at[b], out_hbm.at[pl.ds(base + j * _K, _K)], osem[b]
                )
            return carry

        lax.fori_loop(0, _NGRP, body, 0)
        for b in range(_NBUF):
            wait_out(b)

    return gather_kernel


_gather = _make_gather()


def kernel(wordids, table):
    idx = wordids.reshape(_NW, _NCHUNK, _K)
    if idx.dtype != jnp.int32:
        idx = idx.astype(jnp.int32)
    out = _gather(idx, table)
    return out.reshape(_BATCH, _HIST, _DIM)
